# baseline (speedup 1.0000x reference)
import functools

import jax
import jax.numpy as jnp
from jax import lax
from jax.experimental import pallas as pl
from jax.experimental.pallas import tpu as pltpu
from jax.experimental.pallas import tpu_sc as plsc

N_NODES = 100000
MEM_DIM = 128
BATCH = 16384

NUM_CORES = 2
NUM_SUBCORES = 16
NUM_WORKERS = NUM_CORES * NUM_SUBCORES
ROWS_PER_W = N_NODES // NUM_WORKERS
WPAD = ((ROWS_PER_W + 15) // 16) * 16
NVREG_B = BATCH // 16
NVREG_W = WPAD // 16
NV2 = NVREG_B // 2
JBITS = 14
SENT = 1 << 26
HCAP = BATCH // 2 + 16

NBUF = 5
CHUNK = 125
NCH = ROWS_PER_W // CHUNK
LOOKAHEAD = 3
SEG = -(-NV2 // NCH)


def _body(mem_hbm, val_hbm, idx_hbm, out_hbm,
          idx_v, cande_v, cando_v, winner_v, nlist_v, vlist_v, rowbuf_v,
          buf_v, in_sems, out_sems, gs_sem):
    c = lax.axis_index("c")
    s = lax.axis_index("s")
    wid = s * NUM_CORES + c
    lo = wid * ROWS_PER_W

    lanes = lax.iota(jnp.int32, 16)
    sent_vec = jnp.full((16,), SENT, jnp.int32)
    neg1 = jnp.full((16,), -1, jnp.int32)

    def filt_body(t, offs):
        off_e, off_o = offs
        iv_e = idx_v[pl.ds(t * 32, 16)]
        iv_o = idx_v[pl.ds(t * 32 + 16, 16)]
        rel_e = iv_e - lo
        rel_o = iv_o - lo
        inr_e = rel_e.astype(jnp.uint32) < jnp.uint32(ROWS_PER_W)
        inr_o = rel_o.astype(jnp.uint32) < jnp.uint32(ROWS_PER_W)
        comp_e = (rel_e << JBITS) | (t * 32 + lanes)
        comp_o = (rel_o << JBITS) | (t * 32 + 16 + lanes)
        plsc.store_compressed(cande_v.at[pl.ds(off_e, 16)], comp_e,
                              mask=inr_e)
        plsc.store_compressed(cando_v.at[pl.ds(off_o, 16)], comp_o,
                              mask=inr_o)
        return (off_e + plsc.all_reduce_population_count(inr_e)[0],
                off_o + plsc.all_reduce_population_count(inr_o)[0])

    def in_desc(b, ch):
        return pltpu.make_async_copy(
            mem_hbm.at[pl.ds(lo + ch * CHUNK, CHUNK)],
            buf_v.at[b], in_sems[b])

    def out_desc(b, ch):
        return pltpu.make_async_copy(
            buf_v.at[b],
            out_hbm.at[pl.ds(lo + ch * CHUNK, CHUNK)], out_sems[b])

    for p in range(LOOKAHEAD):
        in_desc(p % NBUF, p).start()

    pltpu.sync_copy(idx_hbm, idx_v)

    offs = (jnp.int32(0), jnp.int32(0))
    for ch in range(NCH):
        la = ch + LOOKAHEAD
        if la < NCH:
            b2 = la % NBUF
            if la >= NBUF:
                out_desc(b2, la - NBUF).wait()
            in_desc(b2, la).start()

        lo_t, hi_t = ch * SEG, min((ch + 1) * SEG, NV2)
        if lo_t < hi_t:
            offs = lax.fori_loop(lo_t, hi_t, filt_body, offs, unroll=4)

        b = ch % NBUF
        in_desc(b, ch).wait()
        out_desc(b, ch).start()

    ncand_e, ncand_o = offs
    cande_v[pl.ds(ncand_e, 16)] = sent_vec
    cando_v[pl.ds(ncand_o, 16)] = sent_vec

    def init_body(k, carry):
        winner_v[pl.ds(k * 16, 16)] = neg1
        return carry

    lax.fori_loop(0, NVREG_W, init_body, 0, unroll=4)

    def sort_keep(cand_ref, t):
        comp = lax.sort(cand_ref[pl.ds(t * 16, 16)])
        nxt = comp.at[jnp.minimum(lanes + 1, 15)].get(
            mode="promise_in_bounds")
        nxt = jnp.where(lanes < 15, nxt, SENT - 1)
        f = comp >> JBITS
        keep = (comp < SENT) & (f != (nxt >> JBITS))
        tgt = jnp.where(keep, f, 0)
        return comp & (BATCH - 1), tgt, keep

    def dedup_e(t, carry):
        jv, tgt, keep = sort_keep(cande_v, t)
        plsc.store_scatter(winner_v, [tgt], jv, mask=keep)
        return carry

    lax.fori_loop(0, (ncand_e + 15) // 16, dedup_e, 0)

    def dedup_o(t, carry):
        jv, tgt, keep = sort_keep(cando_v, t)
        cur = plsc.load_gather(winner_v, [tgt], mask=keep)
        m2 = keep & (jv > cur)
        plsc.store_scatter(winner_v, [tgt], jv, mask=m2)
        return carry

    lax.fori_loop(0, (ncand_o + 15) // 16, dedup_o, 0)

    def comp_body(k, off):
        wv = winner_v[pl.ds(k * 16, 16)]
        m = wv >= 0
        nodes = lo + k * 16 + lanes
        plsc.store_compressed(nlist_v.at[pl.ds(off, 16)], nodes, mask=m)
        plsc.store_compressed(vlist_v.at[pl.ds(off, 16)], wv, mask=m)
        return off + plsc.all_reduce_population_count(m)[0]

    total = lax.fori_loop(0, NVREG_W, comp_body, jnp.int32(0), unroll=4)

    for ch in range(NCH - NBUF, NCH):
        out_desc(ch % NBUF, ch).wait()

    zero16 = jnp.zeros((16,), jnp.int32)

    def emit(nv, vv):
        g = pltpu.make_async_copy(val_hbm.at[vv], rowbuf_v, gs_sem)
        g.start()
        g.wait()
        sct = pltpu.make_async_copy(rowbuf_v, out_hbm.at[nv], gs_sem)
        sct.start()
        sct.wait()

    nfull = total // 16

    def scat_body(cidx, carry):
        nv = nlist_v[pl.ds(cidx * 16, 16)]
        vv = vlist_v[pl.ds(cidx * 16, 16)]
        emit(nv, vv)
        return carry

    lax.fori_loop(0, nfull, scat_body, 0)

    rem = total - nfull * 16

    @pl.when(rem > 0)
    def _():
        nv = nlist_v[pl.ds(nfull * 16, 16)]
        vv = vlist_v[pl.ds(nfull * 16, 16)]
        tm = lanes < rem
        nv0 = nv.at[zero16].get(mode="promise_in_bounds")
        vv0 = vv.at[zero16].get(mode="promise_in_bounds")
        emit(jnp.where(tm, nv, nv0), jnp.where(tm, vv, vv0))


_mesh = plsc.VectorSubcoreMesh(core_axis_name="c", subcore_axis_name="s")

_sc_set = pl.kernel(
    _body,
    out_type=jax.ShapeDtypeStruct((N_NODES, MEM_DIM), jnp.float32),
    mesh=_mesh,
    compiler_params=pltpu.CompilerParams(use_tc_tiling_on_sc=False,
                                         needs_layout_passes=False),
    scratch_types=[
        pltpu.VMEM((BATCH,), jnp.int32),
        pltpu.VMEM((HCAP,), jnp.int32),
        pltpu.VMEM((HCAP,), jnp.int32),
        pltpu.VMEM((WPAD,), jnp.int32),
        pltpu.VMEM((WPAD + 16,), jnp.int32),
        pltpu.VMEM((WPAD + 16,), jnp.int32),
        pltpu.VMEM((16, MEM_DIM), jnp.float32),
        pltpu.VMEM((NBUF, CHUNK, MEM_DIM), jnp.float32),
        [pltpu.SemaphoreType.DMA] * NBUF,
        [pltpu.SemaphoreType.DMA] * NBUF,
        pltpu.SemaphoreType.DMA,
    ],
)


def kernel(memory, values, node_idxs):
    return _sc_set(memory, values, node_idxs.astype(jnp.int32))


# speedup vs baseline: 1.0317x; 1.0317x over previous
import functools

import jax
import jax.numpy as jnp
from jax import lax
from jax.experimental import pallas as pl
from jax.experimental.pallas import tpu as pltpu
from jax.experimental.pallas import tpu_sc as plsc

N_NODES = 100000
MEM_DIM = 128
BATCH = 16384

NUM_CORES = 2
NUM_SUBCORES = 16
NUM_WORKERS = NUM_CORES * NUM_SUBCORES
ROWS_PER_W = N_NODES // NUM_WORKERS
WPAD = ((ROWS_PER_W + 15) // 16) * 16
NVREG_B = BATCH // 16
NVREG_W = WPAD // 16
NV2 = NVREG_B // 2
JBITS = 14
SENT = 1 << 26
HCAP = BATCH // 2 + 16

NBUF = 5
CHUNK = 125
NCH = ROWS_PER_W // CHUNK
LOOKAHEAD = 3
SEG = -(-NV2 // NCH)


def _body(mem_hbm, val_hbm, idx_hbm, out_hbm,
          idx_v, cande_v, cando_v, winner_v, nlist_v, vlist_v, rowbuf_v,
          buf_v, in_sems, out_sems, gs_sem):
    c = lax.axis_index("c")
    s = lax.axis_index("s")
    wid = s * NUM_CORES + c
    lo = wid * ROWS_PER_W

    lanes = lax.iota(jnp.int32, 16)
    sent_vec = jnp.full((16,), SENT, jnp.int32)
    neg1 = jnp.full((16,), -1, jnp.int32)

    def filt_body(t, offs):
        off_e, off_o = offs
        iv_e = idx_v[pl.ds(t * 32, 16)]
        iv_o = idx_v[pl.ds(t * 32 + 16, 16)]
        rel_e = iv_e - lo
        rel_o = iv_o - lo
        inr_e = rel_e.astype(jnp.uint32) < jnp.uint32(ROWS_PER_W)
        inr_o = rel_o.astype(jnp.uint32) < jnp.uint32(ROWS_PER_W)
        comp_e = (rel_e << JBITS) | (t * 32 + lanes)
        comp_o = (rel_o << JBITS) | (t * 32 + 16 + lanes)
        plsc.store_compressed(cande_v.at[pl.ds(off_e, 16)], comp_e,
                              mask=inr_e)
        plsc.store_compressed(cando_v.at[pl.ds(off_o, 16)], comp_o,
                              mask=inr_o)
        return (off_e + plsc.all_reduce_population_count(inr_e)[0],
                off_o + plsc.all_reduce_population_count(inr_o)[0])

    def in_desc(b, ch):
        return pltpu.make_async_copy(
            mem_hbm.at[pl.ds(lo + ch * CHUNK, CHUNK)],
            buf_v.at[b], in_sems[b])

    def out_desc(b, ch):
        return pltpu.make_async_copy(
            buf_v.at[b],
            out_hbm.at[pl.ds(lo + ch * CHUNK, CHUNK)], out_sems[b])

    for p in range(LOOKAHEAD):
        in_desc(p % NBUF, p).start()

    pltpu.sync_copy(idx_hbm, idx_v)

    def init_body(k, carry):
        winner_v[pl.ds(k * 16, 16)] = neg1
        return carry

    def sort_keep(cand_ref, t):
        comp = lax.sort(cand_ref[pl.ds(t * 16, 16)])
        nxt = comp.at[jnp.minimum(lanes + 1, 15)].get(
            mode="promise_in_bounds")
        nxt = jnp.where(lanes < 15, nxt, SENT - 1)
        f = comp >> JBITS
        keep = (comp < SENT) & (f != (nxt >> JBITS))
        tgt = jnp.where(keep, f, 0)
        return comp & (BATCH - 1), tgt, keep

    def dedup_e(t, carry):
        jv, tgt, keep = sort_keep(cande_v, t)
        plsc.store_scatter(winner_v, [tgt], jv, mask=keep)
        return carry

    def dedup_o(t, carry):
        jv, tgt, keep = sort_keep(cando_v, t)
        cur = plsc.load_gather(winner_v, [tgt], mask=keep)
        m2 = keep & (jv > cur)
        plsc.store_scatter(winner_v, [tgt], jv, mask=m2)
        return carry

    def comp_body(k, off):
        wv = winner_v[pl.ds(k * 16, 16)]
        m = wv >= 0
        nodes = lo + k * 16 + lanes
        plsc.store_compressed(nlist_v.at[pl.ds(off, 16)], nodes, mask=m)
        plsc.store_compressed(vlist_v.at[pl.ds(off, 16)], wv, mask=m)
        return off + plsc.all_reduce_population_count(m)[0]

    FSEG = -(-NV2 // 15)
    CSEG = -(-NVREG_W // 5)

    offs = (jnp.int32(0), jnp.int32(0))
    ncand_e = ncand_o = trips_e = trips_o = None
    total = jnp.int32(0)
    for ch in range(NCH):
        la = ch + LOOKAHEAD
        if la < NCH:
            b2 = la % NBUF
            if la >= NBUF:
                out_desc(b2, la - NBUF).wait()
            in_desc(b2, la).start()

        if ch < 15:
            lo_t, hi_t = ch * FSEG, min((ch + 1) * FSEG, NV2)
            if lo_t < hi_t:
                offs = lax.fori_loop(lo_t, hi_t, filt_body, offs, unroll=4)
        elif ch == 15:
            ncand_e, ncand_o = offs
            cande_v[pl.ds(ncand_e, 16)] = sent_vec
            cando_v[pl.ds(ncand_o, 16)] = sent_vec
            lax.fori_loop(0, NVREG_W, init_body, 0, unroll=4)
            trips_e = (ncand_e + 15) // 16
            trips_o = (ncand_o + 15) // 16
        elif ch < 18:
            k = ch - 16
            half = (trips_e + 1) // 2
            lax.fori_loop(k * half, jnp.minimum((k + 1) * half, trips_e),
                          dedup_e, 0)
        elif ch < 20:
            k = ch - 18
            half = (trips_o + 1) // 2
            lax.fori_loop(k * half, jnp.minimum((k + 1) * half, trips_o),
                          dedup_o, 0)
        else:
            k = ch - 20
            total = lax.fori_loop(k * CSEG, min((k + 1) * CSEG, NVREG_W),
                                  comp_body, total, unroll=4)

        b = ch % NBUF
        in_desc(b, ch).wait()
        out_desc(b, ch).start()

    for ch in range(NCH - NBUF, NCH):
        out_desc(ch % NBUF, ch).wait()

    zero16 = jnp.zeros((16,), jnp.int32)

    def emit(nv, vv):
        g = pltpu.make_async_copy(val_hbm.at[vv], rowbuf_v, gs_sem)
        g.start()
        g.wait()
        sct = pltpu.make_async_copy(rowbuf_v, out_hbm.at[nv], gs_sem)
        sct.start()
        sct.wait()

    nfull = total // 16

    def scat_body(cidx, carry):
        nv = nlist_v[pl.ds(cidx * 16, 16)]
        vv = vlist_v[pl.ds(cidx * 16, 16)]
        emit(nv, vv)
        return carry

    lax.fori_loop(0, nfull, scat_body, 0)

    rem = total - nfull * 16

    @pl.when(rem > 0)
    def _():
        nv = nlist_v[pl.ds(nfull * 16, 16)]
        vv = vlist_v[pl.ds(nfull * 16, 16)]
        tm = lanes < rem
        nv0 = nv.at[zero16].get(mode="promise_in_bounds")
        vv0 = vv.at[zero16].get(mode="promise_in_bounds")
        emit(jnp.where(tm, nv, nv0), jnp.where(tm, vv, vv0))


_mesh = plsc.VectorSubcoreMesh(core_axis_name="c", subcore_axis_name="s")

_sc_set = pl.kernel(
    _body,
    out_type=jax.ShapeDtypeStruct((N_NODES, MEM_DIM), jnp.float32),
    mesh=_mesh,
    compiler_params=pltpu.CompilerParams(use_tc_tiling_on_sc=False,
                                         needs_layout_passes=False),
    scratch_types=[
        pltpu.VMEM((BATCH,), jnp.int32),
        pltpu.VMEM((HCAP,), jnp.int32),
        pltpu.VMEM((HCAP,), jnp.int32),
        pltpu.VMEM((WPAD,), jnp.int32),
        pltpu.VMEM((WPAD + 16,), jnp.int32),
        pltpu.VMEM((WPAD + 16,), jnp.int32),
        pltpu.VMEM((16, MEM_DIM), jnp.float32),
        pltpu.VMEM((NBUF, CHUNK, MEM_DIM), jnp.float32),
        [pltpu.SemaphoreType.DMA] * NBUF,
        [pltpu.SemaphoreType.DMA] * NBUF,
        pltpu.SemaphoreType.DMA,
    ],
)


def kernel(memory, values, node_idxs):
    return _sc_set(memory, values, node_idxs.astype(jnp.int32))


# speedup vs baseline: 1.2656x; 1.2266x over previous
import functools

import jax
import jax.numpy as jnp
from jax import lax
from jax.experimental import pallas as pl
from jax.experimental.pallas import tpu as pltpu
from jax.experimental.pallas import tpu_sc as plsc

N_NODES = 100000
MEM_DIM = 128
BATCH = 16384

NUM_CORES = 2
NUM_SUBCORES = 16
NUM_WORKERS = NUM_CORES * NUM_SUBCORES
ROWS_PER_W = N_NODES // NUM_WORKERS
WPAD = ((ROWS_PER_W + 15) // 16) * 16
NVREG_B = BATCH // 16
NVREG_W = WPAD // 16
NV2 = NVREG_B // 2
JBITS = 14
SENT = 1 << 26
HCAP = BATCH // 2 + 16

NBUF = 4
CHUNK = 125
NCH = ROWS_PER_W // CHUNK
LOOKAHEAD = 2
SEG = -(-NV2 // NCH)


def _body(mem_hbm, val_hbm, idx_hbm, out_hbm,
          idx_v, cande_v, cando_v, winner_v, nlist_v, vlist_v, rowbuf_v,
          buf_v, in_sems, out_sems, gsems, ssems):
    c = lax.axis_index("c")
    s = lax.axis_index("s")
    wid = s * NUM_CORES + c
    lo = wid * ROWS_PER_W

    lanes = lax.iota(jnp.int32, 16)
    sent_vec = jnp.full((16,), SENT, jnp.int32)
    neg1 = jnp.full((16,), -1, jnp.int32)

    def filt_body(t, offs):
        off_e, off_o = offs
        iv_e = idx_v[pl.ds(t * 32, 16)]
        iv_o = idx_v[pl.ds(t * 32 + 16, 16)]
        rel_e = iv_e - lo
        rel_o = iv_o - lo
        inr_e = rel_e.astype(jnp.uint32) < jnp.uint32(ROWS_PER_W)
        inr_o = rel_o.astype(jnp.uint32) < jnp.uint32(ROWS_PER_W)
        comp_e = (rel_e << JBITS) | (t * 32 + lanes)
        comp_o = (rel_o << JBITS) | (t * 32 + 16 + lanes)
        plsc.store_compressed(cande_v.at[pl.ds(off_e, 16)], comp_e,
                              mask=inr_e)
        plsc.store_compressed(cando_v.at[pl.ds(off_o, 16)], comp_o,
                              mask=inr_o)
        return (off_e + plsc.all_reduce_population_count(inr_e)[0],
                off_o + plsc.all_reduce_population_count(inr_o)[0])

    def in_desc(b, ch):
        return pltpu.make_async_copy(
            mem_hbm.at[pl.ds(lo + ch * CHUNK, CHUNK)],
            buf_v.at[b], in_sems[b])

    def out_desc(b, ch):
        return pltpu.make_async_copy(
            buf_v.at[b],
            out_hbm.at[pl.ds(lo + ch * CHUNK, CHUNK)], out_sems[b])

    for p in range(LOOKAHEAD):
        in_desc(p % NBUF, p).start()

    pltpu.sync_copy(idx_hbm, idx_v)

    def init_body(k, carry):
        winner_v[pl.ds(k * 16, 16)] = neg1
        return carry

    def sort_keep(cand_ref, t):
        comp = lax.sort(cand_ref[pl.ds(t * 16, 16)])
        nxt = comp.at[jnp.minimum(lanes + 1, 15)].get(
            mode="promise_in_bounds")
        nxt = jnp.where(lanes < 15, nxt, SENT - 1)
        f = comp >> JBITS
        keep = (comp < SENT) & (f != (nxt >> JBITS))
        tgt = jnp.where(keep, f, 0)
        return comp & (BATCH - 1), tgt, keep

    def dedup_e(t, carry):
        jv, tgt, keep = sort_keep(cande_v, t)
        plsc.store_scatter(winner_v, [tgt], jv, mask=keep)
        return carry

    def dedup_o(t, carry):
        jv, tgt, keep = sort_keep(cando_v, t)
        cur = plsc.load_gather(winner_v, [tgt], mask=keep)
        m2 = keep & (jv > cur)
        plsc.store_scatter(winner_v, [tgt], jv, mask=m2)
        return carry

    def comp_body(k, off):
        wv = winner_v[pl.ds(k * 16, 16)]
        m = wv >= 0
        nodes = lo + k * 16 + lanes
        plsc.store_compressed(nlist_v.at[pl.ds(off, 16)], nodes, mask=m)
        plsc.store_compressed(vlist_v.at[pl.ds(off, 16)], wv, mask=m)
        return off + plsc.all_reduce_population_count(m)[0]

    FSEG = -(-NV2 // 15)
    CSEG = -(-NVREG_W // 5)

    offs = (jnp.int32(0), jnp.int32(0))
    ncand_e = ncand_o = trips_e = trips_o = None
    total = jnp.int32(0)
    for ch in range(NCH):
        la = ch + LOOKAHEAD
        if la < NCH:
            b2 = la % NBUF
            if la >= NBUF:
                out_desc(b2, la - NBUF).wait()
            in_desc(b2, la).start()

        if ch < 15:
            lo_t, hi_t = ch * FSEG, min((ch + 1) * FSEG, NV2)
            if lo_t < hi_t:
                offs = lax.fori_loop(lo_t, hi_t, filt_body, offs, unroll=4)
        elif ch == 15:
            ncand_e, ncand_o = offs
            cande_v[pl.ds(ncand_e, 16)] = sent_vec
            cando_v[pl.ds(ncand_o, 16)] = sent_vec
            lax.fori_loop(0, NVREG_W, init_body, 0, unroll=4)
            trips_e = (ncand_e + 15) // 16
            trips_o = (ncand_o + 15) // 16
        elif ch < 18:
            k = ch - 16
            half = (trips_e + 1) // 2
            lax.fori_loop(k * half, jnp.minimum((k + 1) * half, trips_e),
                          dedup_e, 0)
        elif ch < 20:
            k = ch - 18
            half = (trips_o + 1) // 2
            lax.fori_loop(k * half, jnp.minimum((k + 1) * half, trips_o),
                          dedup_o, 0)
        else:
            k = ch - 20
            total = lax.fori_loop(k * CSEG, min((k + 1) * CSEG, NVREG_W),
                                  comp_body, total, unroll=4)

        b = ch % NBUF
        in_desc(b, ch).wait()
        out_desc(b, ch).start()

    for ch in range(NCH - NBUF, NCH):
        out_desc(ch % NBUF, ch).wait()

    zero16 = jnp.zeros((16,), jnp.int32)
    nfull = total // 16
    rem = total - nfull * 16

    @pl.when(rem > 0)
    def _():
        nv = nlist_v[pl.ds(nfull * 16, 16)]
        vv = vlist_v[pl.ds(nfull * 16, 16)]
        tm = lanes < rem
        nv0 = nv.at[zero16].get(mode="promise_in_bounds")
        vv0 = vv.at[zero16].get(mode="promise_in_bounds")
        nlist_v[pl.ds(nfull * 16, 16)] = jnp.where(tm, nv, nv0)
        vlist_v[pl.ds(nfull * 16, 16)] = jnp.where(tm, vv, vv0)

    nchunks = (total + 15) // 16

    def start_g(ci, b):
        vv = vlist_v[pl.ds(ci * 16, 16)]
        pltpu.make_async_copy(val_hbm.at[vv], rowbuf_v.at[b], gsems[b]).start()

    def wait_g(b):
        pltpu.make_async_copy(val_hbm.at[zero16], rowbuf_v.at[b],
                              gsems[b]).wait()

    def start_s(ci, b):
        nv = nlist_v[pl.ds(ci * 16, 16)]
        pltpu.make_async_copy(rowbuf_v.at[b], out_hbm.at[nv], ssems[b]).start()

    def drain_s(b):
        pltpu.make_async_copy(rowbuf_v.at[b], out_hbm.at[zero16],
                              ssems[b]).wait()

    @pl.when(nchunks > 0)
    def _():
        start_g(0, 0)

    @pl.when(nchunks > 1)
    def _():
        start_g(1, 1)

    def _step(ci, b):
        b2 = (b + 2) % 4

        @pl.when(ci >= 2)
        def _():
            drain_s(b2)

        @pl.when(ci + 2 < nchunks)
        def _():
            start_g(ci + 2, b2)

        wait_g(b)
        start_s(ci, b)

    def pipe_body(ci, carry):
        for bb in range(4):
            @pl.when(ci % 4 == bb)
            def _(bb=bb):
                _step(ci, bb)
        return carry

    lax.fori_loop(0, nchunks, pipe_body, 0)

    for bb in range(4):
        @pl.when((nchunks >= 1) & ((nchunks - 1) % 4 == bb))
        def _(bb=bb):
            drain_s(bb)

        @pl.when((nchunks >= 2) & ((nchunks - 2) % 4 == bb))
        def _(bb=bb):
            drain_s(bb)


_mesh = plsc.VectorSubcoreMesh(core_axis_name="c", subcore_axis_name="s")

_sc_set = pl.kernel(
    _body,
    out_type=jax.ShapeDtypeStruct((N_NODES, MEM_DIM), jnp.float32),
    mesh=_mesh,
    compiler_params=pltpu.CompilerParams(use_tc_tiling_on_sc=False,
                                         needs_layout_passes=False),
    scratch_types=[
        pltpu.VMEM((BATCH,), jnp.int32),
        pltpu.VMEM((HCAP,), jnp.int32),
        pltpu.VMEM((HCAP,), jnp.int32),
        pltpu.VMEM((WPAD,), jnp.int32),
        pltpu.VMEM((WPAD + 16,), jnp.int32),
        pltpu.VMEM((WPAD + 16,), jnp.int32),
        pltpu.VMEM((4, 16, MEM_DIM), jnp.float32),
        pltpu.VMEM((NBUF, CHUNK, MEM_DIM), jnp.float32),
        [pltpu.SemaphoreType.DMA] * NBUF,
        [pltpu.SemaphoreType.DMA] * NBUF,
        [pltpu.SemaphoreType.DMA] * 4,
        [pltpu.SemaphoreType.DMA] * 4,
    ],
)


def kernel(memory, values, node_idxs):
    return _sc_set(memory, values, node_idxs.astype(jnp.int32))
